# Initial kernel scaffold; baseline (speedup 1.0000x reference)
import jax
import jax.numpy as jnp
from jax.experimental import pallas as pl


def kernel(x, edge_index, edge_attr, lin0_W, lin0_b, nn_W, nn_b, root_W, conv_b):
    raise NotImplementedError("write your pallas kernel here")


# speedup vs baseline: 2.4767x; 2.4767x over previous
import jax
import jax.numpy as jnp
from jax import lax
from jax.experimental import pallas as pl
from jax.experimental.pallas import tpu as pltpu
from jax.experimental.pallas import tpu_sc as plsc

H = 16
KC = 5
NC = 2
NS = 16
NW = NC * NS
CHUNK = 128
PW = 128


def _edge_pass_kernel(n_pad: int, n_chunks_per_w: int):
    rows_per_tile = n_pad // NS

    def body(p_hbm, src_hbm, dst_hbm, coef_hbm, out_hbm,
             src_v, dst_v, coef_v, g_v, msg_v, agg_sh, sem):
        c = lax.axis_index("c")
        s = lax.axis_index("s")
        wid = s * NC + c

        def zrow(i, _):
            for j in range(PW // H):
                msg_v[i, pl.ds(j * H, H)] = jnp.zeros((H,), jnp.float32)
            return 0
        lax.fori_loop(0, CHUNK, zrow, 0)
        for jj in range(rows_per_tile // CHUNK):
            pltpu.sync_copy(
                msg_v, agg_sh.at[pl.ds(s * rows_per_tile + jj * CHUNK, CHUNK)])
        plsc.subcore_barrier()

        def chunk_body(i, _):
            cid = wid * n_chunks_per_w + i
            pltpu.sync_copy(src_hbm.at[pl.ds(cid * CHUNK, CHUNK)], src_v)
            pltpu.sync_copy(dst_hbm.at[pl.ds(cid * CHUNK, CHUNK)], dst_v)
            pltpu.sync_copy(coef_hbm.at[pl.ds(cid * (CHUNK * H), CHUNK * H)],
                            coef_v)
            pltpu.async_copy(p_hbm.at[src_v], g_v, sem).wait()

            def edge_body(e2, _):
                cvec = coef_v[pl.ds(e2 * H, H)]
                acc = cvec[0] * g_v[e2, pl.ds(0, H)]
                for k in range(1, KC):
                    acc = acc + cvec[k] * g_v[e2, pl.ds(k * H, H)]
                msg_v[e2, pl.ds(0, H)] = acc
                return 0
            lax.fori_loop(0, CHUNK, edge_body, 0, unroll=4)
            pltpu.sync_copy(msg_v, agg_sh.at[dst_v], add=True)
            return 0
        lax.fori_loop(0, n_chunks_per_w, chunk_body, 0)

        plsc.subcore_barrier()
        pltpu.sync_copy(
            agg_sh.at[pl.ds(s * rows_per_tile, rows_per_tile)],
            out_hbm.at[c, pl.ds(s * rows_per_tile, rows_per_tile)])

    mesh = plsc.VectorSubcoreMesh(core_axis_name="c", subcore_axis_name="s")
    return pl.kernel(
        body,
        out_type=jax.ShapeDtypeStruct((NC, n_pad, PW), jnp.float32),
        mesh=mesh,
        scratch_types=[
            pltpu.VMEM((CHUNK,), jnp.int32),
            pltpu.VMEM((CHUNK,), jnp.int32),
            pltpu.VMEM((CHUNK * H,), jnp.float32),
            pltpu.VMEM((CHUNK, PW), jnp.float32),
            pltpu.VMEM((CHUNK, PW), jnp.float32),
            pltpu.VMEM_SHARED((n_pad, PW), jnp.float32),
            pltpu.SemaphoreType.DMA,
        ],
    )


def _tc_first(x_ref, w0_ref, b0_ref, wcat_ref, root_ref,
              out_ref, p_ref, r_ref):
    out = jnp.maximum(
        jnp.dot(x_ref[...], w0_ref[...],
                preferred_element_type=jnp.float32, precision=jax.lax.Precision.HIGHEST) + b0_ref[...], 0.0)
    out_ref[...] = out
    p_ref[...] = jnp.dot(out, wcat_ref[...], preferred_element_type=jnp.float32, precision=jax.lax.Precision.HIGHEST)
    r_ref[...] = jnp.dot(out, root_ref[...], preferred_element_type=jnp.float32, precision=jax.lax.Precision.HIGHEST)


def _make_tc_mid(n: int):
    def _tc_mid(agg_ref, r_ref, cb_ref, wcat_ref, root_ref,
                out_ref, p_ref, rn_ref):
        out = jnp.maximum(
            agg_ref[0, :n, :H] + agg_ref[1, :n, :H] + r_ref[...] + cb_ref[...],
            0.0)
        out_ref[...] = out
        p_ref[...] = jnp.dot(out, wcat_ref[...],
                             preferred_element_type=jnp.float32,
                             precision=jax.lax.Precision.HIGHEST)
        rn_ref[...] = jnp.dot(out, root_ref[...],
                              preferred_element_type=jnp.float32,
                              precision=jax.lax.Precision.HIGHEST)
    return _tc_mid


def _make_tc_last(n: int):
    def _tc_last(agg_ref, r_ref, cb_ref, out_ref):
        out_ref[...] = jnp.maximum(
            agg_ref[0, :n, :H] + agg_ref[1, :n, :H] + r_ref[...] + cb_ref[...],
            0.0)
    return _tc_last


def kernel(x, edge_index, edge_attr, lin0_W, lin0_b, nn_W, nn_b, root_W, conv_b):
    n, _ = x.shape
    e = edge_index.shape[1]
    n_pad = -(-n // (NS * CHUNK)) * (NS * CHUNK)

    wcat = jnp.concatenate(
        [nn_W.reshape(4, H, H).transpose(1, 0, 2).reshape(H, 4 * H),
         nn_b.reshape(H, H),
         jnp.zeros((H, PW - KC * H), jnp.float32)], axis=1)

    per_w = -(-e // (NW * CHUNK)) * CHUNK
    e_pad = per_w * NW
    n_chunks_per_w = per_w // CHUNK
    pad = e_pad - e
    src = jnp.pad(edge_index[0], (0, pad))
    dst = jnp.pad(edge_index[1], (0, pad))
    coef = jnp.concatenate(
        [edge_attr, jnp.ones((e, 1), jnp.float32),
         jnp.zeros((e, H - KC), jnp.float32)], axis=1)
    coef = jnp.pad(coef, ((0, pad), (0, 0))).reshape(-1)

    b0 = lin0_b.reshape(1, H)
    cb = conv_b.reshape(1, H)

    out, p, r = pl.pallas_call(
        _tc_first,
        out_shape=[
            jax.ShapeDtypeStruct((n, H), jnp.float32),
            jax.ShapeDtypeStruct((n, PW), jnp.float32),
            jax.ShapeDtypeStruct((n, H), jnp.float32),
        ],
    )(x, lin0_W, b0, wcat, root_W)

    edge_pass = _edge_pass_kernel(n_pad, n_chunks_per_w)
    tc_mid = _make_tc_mid(n)
    tc_last = _make_tc_last(n)

    for layer in range(3):
        agg2 = edge_pass(p, src, dst, coef)
        if layer < 2:
            out, p, r = pl.pallas_call(
                tc_mid,
                out_shape=[
                    jax.ShapeDtypeStruct((n, H), jnp.float32),
                    jax.ShapeDtypeStruct((n, PW), jnp.float32),
                    jax.ShapeDtypeStruct((n, H), jnp.float32),
                ],
            )(agg2, r, cb, wcat, root_W)
        else:
            out = pl.pallas_call(
                tc_last,
                out_shape=jax.ShapeDtypeStruct((n, H), jnp.float32),
            )(agg2, r, cb)
    return out
